# Initial kernel scaffold; baseline (speedup 1.0000x reference)
import jax
import jax.numpy as jnp
from jax.experimental import pallas as pl


def kernel(x, Wq, bq, Wk, bk, Wv, bv, Wo, bo, ln1_w, ln1_b, ln2_w, ln2_b, gate_W, gate_b, W1, b1, W2, b2):
    raise NotImplementedError("write your pallas kernel here")


# speedup vs baseline: 2.6523x; 2.6523x over previous
import functools
import math

import jax
import jax.numpy as jnp
from jax import lax
from jax.experimental import pallas as pl
from jax.experimental.pallas import tpu as pltpu
from jax.experimental.pallas import tpu_sc as plsc

S, D, H, E, K, F = 2048, 768, 12, 8, 2, 2048
DH = D // H
QB = 512
NQB = S // QB
GP = H // 2
TT = 256
NT = (S * K) // TT
G = NT + E - 1
NSC = 32


KC = 1024
QC = 1024


def _qkv_body(x_ref, wq, wk, wv, bq, bk, bv, q_ref, k_ref, v_ref):
    x = x_ref[...]
    q = jnp.dot(x, wq[...], preferred_element_type=jnp.float32) + bq[...]
    k = jnp.dot(x, wk[...], preferred_element_type=jnp.float32) + bk[...]
    v = jnp.dot(x, wv[...], preferred_element_type=jnp.float32) + bv[...]
    q_ref[...] = q.astype(jnp.bfloat16)
    k_ref[...] = k.astype(jnp.bfloat16)
    v_ref[...] = v


def _qkv_proj(x2, Wq, bq, Wk, bk, Wv, bv):
    return pl.pallas_call(
        _qkv_body,
        grid=(NQB,),
        in_specs=[
            pl.BlockSpec((QB, D), lambda i: (i, 0)),
            pl.BlockSpec((D, D), lambda i: (0, 0)),
            pl.BlockSpec((D, D), lambda i: (0, 0)),
            pl.BlockSpec((D, D), lambda i: (0, 0)),
            pl.BlockSpec((1, D), lambda i: (0, 0)),
            pl.BlockSpec((1, D), lambda i: (0, 0)),
            pl.BlockSpec((1, D), lambda i: (0, 0)),
        ],
        out_specs=[
            pl.BlockSpec((QB, D), lambda i: (i, 0)),
            pl.BlockSpec((QB, D), lambda i: (i, 0)),
            pl.BlockSpec((QB, D), lambda i: (i, 0)),
        ],
        out_shape=[
            jax.ShapeDtypeStruct((S, D), jnp.bfloat16),
            jax.ShapeDtypeStruct((S, D), jnp.bfloat16),
            jax.ShapeDtypeStruct((S, D), jnp.float32),
        ],
    )(x2, Wq, Wk, Wv, bq.reshape(1, D), bk.reshape(1, D), bv.reshape(1, D))


def _attn_body(q_ref, k_ref, v_ref, o_ref):
    q = q_ref[...]
    for hh in range(2):
        sl = slice(hh * DH, (hh + 1) * DH)
        s = lax.dot_general(q[:, sl], k_ref[:, sl], (((1,), (1,)), ((), ())),
                            preferred_element_type=jnp.float32) * 0.125
        s0 = s[:, :KC]
        m = jnp.max(s0, axis=-1, keepdims=True)
        ex = jnp.exp(s0 - m)
        ssum = jnp.sum(ex, axis=-1, keepdims=True)
        acc = jnp.dot(ex, v_ref[:KC, sl].astype(jnp.float32),
                      preferred_element_type=jnp.float32)
        s1 = s[:, KC:]
        m1 = jnp.max(s1, axis=-1, keepdims=True)
        m_new = jnp.maximum(m, m1)
        delta = jnp.where(m == m_new, 0.0, m - m_new)
        corr = jnp.exp(delta)
        ex1 = jnp.exp(s1 - m_new)
        ssum = corr * ssum + jnp.sum(ex1, axis=-1, keepdims=True)
        acc = corr * acc + jnp.dot(ex1, v_ref[KC:, sl].astype(jnp.float32),
                                   preferred_element_type=jnp.float32)
        o_ref[:, sl] = (acc * (1.0 / ssum)).astype(jnp.bfloat16)


def _attention(q16, k16, v):
    return pl.pallas_call(
        _attn_body,
        grid=(GP, S // QC),
        in_specs=[
            pl.BlockSpec((QC, 2 * DH), lambda g, qb: (qb, g)),
            pl.BlockSpec((S, 2 * DH), lambda g, qb: (0, g)),
            pl.BlockSpec((S, 2 * DH), lambda g, qb: (0, g)),
        ],
        out_specs=pl.BlockSpec((QC, 2 * DH), lambda g, qb: (qb, g)),
        out_shape=jax.ShapeDtypeStruct((S, D), jnp.bfloat16),
    )(q16, k16, v)


def _proj_ln1_body(o_ref, wo_ref, bo_ref, w_ref, b_ref, x_ref, h_ref):
    a = jnp.dot(o_ref[...].astype(jnp.float32), wo_ref[...],
                preferred_element_type=jnp.float32) + bo_ref[...]
    mu = jnp.sum(a, axis=-1, keepdims=True) * jnp.float32(1.0 / D)
    var = jnp.sum((a - mu) ** 2, axis=-1, keepdims=True) * jnp.float32(1.0 / D)
    y = (a - mu) / jnp.sqrt(var + 1e-5) * w_ref[...] + b_ref[...]
    h_ref[...] = x_ref[...] + y


def _proj_ln1(o, Wo, bo, ln1_w, ln1_b, x2):
    return pl.pallas_call(
        _proj_ln1_body,
        grid=(NQB,),
        in_specs=[
            pl.BlockSpec((QB, D), lambda i: (i, 0)),
            pl.BlockSpec((D, D), lambda i: (0, 0)),
            pl.BlockSpec((1, D), lambda i: (0, 0)),
            pl.BlockSpec((1, D), lambda i: (0, 0)),
            pl.BlockSpec((1, D), lambda i: (0, 0)),
            pl.BlockSpec((QB, D), lambda i: (i, 0)),
        ],
        out_specs=pl.BlockSpec((QB, D), lambda i: (i, 0)),
        out_shape=jax.ShapeDtypeStruct((S, D), jnp.float32),
    )(o, Wo, bo.reshape(1, D), ln1_w.reshape(1, D), ln1_b.reshape(1, D), x2)


def _lane_cumsum_excl(v, n):
    c = v
    sh = 1
    while sh < n:
        c = c + jnp.concatenate([jnp.zeros((1, sh), jnp.float32), c[:, : n - sh]],
                                axis=1)
        sh *= 2
    return c - v


def _route_body(h_ref, gw_ref, gb_ref, pos_ref, meta_ref, vals_ref):
    hh = h_ref[...]
    logits = jnp.dot(hh, gw_ref[...], preferred_element_type=jnp.float32) + gb_ref[...]
    m = jnp.max(logits, axis=-1, keepdims=True)
    ex = jnp.exp(logits - m)
    p = ex / jnp.sum(ex, axis=-1, keepdims=True)
    io8 = lax.broadcasted_iota(jnp.int32, (S, E), 1)
    v1 = jnp.max(p, axis=-1, keepdims=True)
    i1 = jnp.min(jnp.where(p == v1, io8, E), axis=-1, keepdims=True)
    pm = jnp.where(io8 == i1, -jnp.inf, p)
    v2 = jnp.max(pm, axis=-1, keepdims=True)
    i2 = jnp.min(jnp.where(pm == v2, io8, E), axis=-1, keepdims=True)
    vals_ref[...] = jnp.concatenate([v1, v2], axis=1)

    oh0 = (io8 == i1).astype(jnp.float32)
    oh1 = (io8 == i2).astype(jnp.float32)
    X = jnp.concatenate([oh0, oh1], axis=0)

    C = X
    sh = 1
    while sh < S * K:
        z = jnp.zeros((sh, E), jnp.float32)
        C = C + jnp.concatenate([z, C[: S * K - sh]], axis=0)
        sh *= 2

    counts = jnp.sum(X, axis=0, keepdims=True)
    offsets = _lane_cumsum_excl(counts, E)
    rank1 = jnp.sum(X * C, axis=-1, keepdims=True)
    offrow = jnp.sum(X * offsets, axis=-1, keepdims=True)
    pos_ref[...] = (offrow + rank1 - 1.0).astype(jnp.int32)

    start = offsets
    end = offsets + counts
    nonempty = counts > 0.0
    ft = jnp.floor(start / TT)
    lt = jnp.floor((end - 1.0) / TT)
    ntiles = jnp.where(nonempty, lt - ft + 1.0, 0.0)
    cum_excl = _lane_cumsum_excl(ntiles, E)
    cum_incl = cum_excl + ntiles
    total = jnp.sum(ntiles, axis=-1, keepdims=True)

    gio = lax.broadcasted_iota(jnp.int32, (G, 1), 0).astype(jnp.float32)
    e_of_g = jnp.sum((cum_incl <= gio).astype(jnp.float32), axis=-1, keepdims=True)
    valid = gio < total
    e_cl = jnp.minimum(e_of_g, float(E - 1))
    sel = (lax.broadcasted_iota(jnp.int32, (G, E), 1)
           == e_cl.astype(jnp.int32)).astype(jnp.float32)
    ft_g = jnp.sum(sel * ft, axis=-1, keepdims=True)
    ce_g = jnp.sum(sel * cum_excl, axis=-1, keepdims=True)
    st_g = jnp.sum(sel * start, axis=-1, keepdims=True)
    en_g = jnp.sum(sel * end, axis=-1, keepdims=True)
    tile_g = jnp.where(valid, ft_g + (gio - ce_g), float(NT - 1))
    lo_g = jnp.where(valid, jnp.clip(st_g - tile_g * TT, 0.0, float(TT)), 0.0)
    hi_g = jnp.where(valid, jnp.clip(en_g - tile_g * TT, 0.0, float(TT)), 0.0)
    zg = jnp.zeros((G, 1), jnp.float32)
    meta = jnp.concatenate(
        [tile_g, e_cl, lo_g, hi_g, zg, zg, zg, zg], axis=1)
    meta_ref[...] = meta.astype(jnp.int32)


def _route(h, gate_W, gate_b):
    return pl.pallas_call(
        _route_body,
        in_specs=[
            pl.BlockSpec((S, D), lambda: (0, 0)),
            pl.BlockSpec((D, E), lambda: (0, 0)),
            pl.BlockSpec((1, E), lambda: (0, 0)),
        ],
        out_specs=[
            pl.BlockSpec((S * K, 1), lambda: (0, 0)),
            pl.BlockSpec((G, 8), lambda: (0, 0)),
            pl.BlockSpec((S, 2), lambda: (0, 0)),
        ],
        out_shape=[
            jax.ShapeDtypeStruct((S * K, 1), jnp.int32),
            jax.ShapeDtypeStruct((G, 8), jnp.int32),
            jax.ShapeDtypeStruct((S, 2), jnp.float32),
        ],
    )(h, gate_W, gate_b.reshape(1, E))


PPW = (S * K) // NSC


def _dispatch_body(h_hbm, pos_hbm, out_hbm, tok_v, pos_v, rows_v, sem):
    wid = lax.axis_index("s") * 2 + lax.axis_index("c")
    base = wid * PPW
    pltpu.sync_copy(pos_hbm.at[pl.ds(base, PPW)], pos_v)
    for c in range(PPW // 16):
        idx = lax.iota(jnp.int32, 16) + (base + c * 16)
        tok_v[pl.ds(c * 16, 16)] = jnp.bitwise_and(idx, S - 1)
    pltpu.async_copy(h_hbm.at[tok_v], rows_v, sem).wait()
    pltpu.async_copy(rows_v, out_hbm.at[pos_v], sem).wait()


def _dispatch(h, pos):
    f = functools.partial(
        pl.kernel,
        out_type=jax.ShapeDtypeStruct((S * K, D), jnp.float32),
        mesh=plsc.VectorSubcoreMesh(core_axis_name="c", subcore_axis_name="s"),
        scratch_types=[
            pltpu.VMEM((PPW,), jnp.int32),
            pltpu.VMEM((PPW,), jnp.int32),
            pltpu.VMEM((PPW, D), jnp.float32),
            pltpu.SemaphoreType.DMA,
        ],
    )(_dispatch_body)
    return f(h, pos)


SQRT_HALF = 1.0 / math.sqrt(2.0)


def _gmm_body(meta_ref, xg_ref, w1_ref, b1_ref, w2_ref, b2_ref, out_ref):
    g = pl.program_id(0)
    tile = meta_ref[g, 0]
    lo = meta_ref[g, 2]
    hi = meta_ref[g, 3]
    prev_tile = meta_ref[jnp.maximum(g - 1, 0), 0]

    @pl.when(jnp.logical_or(g == 0, prev_tile != tile))
    def _():
        out_ref[...] = jnp.zeros_like(out_ref)

    @pl.when(hi > lo)
    def _():
        xg = xg_ref[...]
        w1 = w1_ref[...].reshape(D, F)
        w2 = w2_ref[...].reshape(F, D)
        h1 = jnp.dot(xg, w1, preferred_element_type=jnp.float32) + b1_ref[...].reshape(1, F)
        h1 = h1 * 0.5 * (1.0 + lax.erf(h1 * SQRT_HALF))
        h2 = jnp.dot(h1, w2, preferred_element_type=jnp.float32) + b2_ref[...].reshape(1, D)
        rows = lax.broadcasted_iota(jnp.int32, (TT, 1), 0)
        mask = jnp.logical_and(rows >= lo, rows < hi)
        out_ref[...] += jnp.where(mask, h2, 0.0)


def _gmm(meta, gathered, W1, b1, W2, b2):
    return pl.pallas_call(
        _gmm_body,
        grid_spec=pltpu.PrefetchScalarGridSpec(
            num_scalar_prefetch=1,
            grid=(G,),
            in_specs=[
                pl.BlockSpec((TT, D), lambda g, m: (m[g, 0], 0)),
                pl.BlockSpec((1, D, F), lambda g, m: (m[g, 1], 0, 0)),
                pl.BlockSpec((1, 1, F), lambda g, m: (m[g, 1], 0, 0)),
                pl.BlockSpec((1, F, D), lambda g, m: (m[g, 1], 0, 0)),
                pl.BlockSpec((1, 1, D), lambda g, m: (m[g, 1], 0, 0)),
            ],
            out_specs=pl.BlockSpec((TT, D), lambda g, m: (m[g, 0], 0)),
        ),
        out_shape=jax.ShapeDtypeStruct((S * K, D), jnp.float32),
    )(meta, gathered, W1, b1.reshape(E, 1, F), W2, b2.reshape(E, 1, D))


TPW = S // NSC


def _combine_body(eo_hbm, pos_hbm, out0_hbm, out1_hbm, idx_v, rows_v, sem):
    wid = lax.axis_index("s") * 2 + lax.axis_index("c")
    base = wid * TPW
    pltpu.sync_copy(pos_hbm.at[pl.ds(base, TPW)], idx_v)
    pltpu.async_copy(eo_hbm.at[idx_v], rows_v, sem).wait()
    pltpu.sync_copy(rows_v, out0_hbm.at[pl.ds(base, TPW)])
    pltpu.sync_copy(pos_hbm.at[pl.ds(S + base, TPW)], idx_v)
    pltpu.async_copy(eo_hbm.at[idx_v], rows_v, sem).wait()
    pltpu.sync_copy(rows_v, out1_hbm.at[pl.ds(base, TPW)])


def _combine(eo, pos):
    f = functools.partial(
        pl.kernel,
        out_type=(jax.ShapeDtypeStruct((S, D), jnp.float32),
                  jax.ShapeDtypeStruct((S, D), jnp.float32)),
        mesh=plsc.VectorSubcoreMesh(core_axis_name="c", subcore_axis_name="s"),
        scratch_types=[
            pltpu.VMEM((TPW,), jnp.int32),
            pltpu.VMEM((TPW, D), jnp.float32),
            pltpu.SemaphoreType.DMA,
        ],
    )(_combine_body)
    return f(eo, pos)


def _ln2_body(h_ref, e0_ref, e1_ref, v_ref, w_ref, b_ref, out_ref):
    v = v_ref[...]
    moe = v[:, 0:1] * e0_ref[...] + v[:, 1:2] * e1_ref[...]
    mu = jnp.sum(moe, axis=-1, keepdims=True) * jnp.float32(1.0 / D)
    var = jnp.sum((moe - mu) ** 2, axis=-1, keepdims=True) * jnp.float32(1.0 / D)
    y = (moe - mu) / jnp.sqrt(var + 1e-5) * w_ref[...] + b_ref[...]
    out_ref[...] = h_ref[...] + y


def _ln2(h, eo0, eo1, vals, ln2_w, ln2_b):
    return pl.pallas_call(
        _ln2_body,
        grid=(NQB,),
        in_specs=[
            pl.BlockSpec((QB, D), lambda i: (i, 0)),
            pl.BlockSpec((QB, D), lambda i: (i, 0)),
            pl.BlockSpec((QB, D), lambda i: (i, 0)),
            pl.BlockSpec((QB, 2), lambda i: (i, 0)),
            pl.BlockSpec((1, D), lambda i: (0, 0)),
            pl.BlockSpec((1, D), lambda i: (0, 0)),
        ],
        out_specs=pl.BlockSpec((QB, D), lambda i: (i, 0)),
        out_shape=jax.ShapeDtypeStruct((S, D), jnp.float32),
    )(h, eo0, eo1, vals, ln2_w.reshape(1, D), ln2_b.reshape(1, D))


def kernel(x, Wq, bq, Wk, bk, Wv, bv, Wo, bo, ln1_w, ln1_b, ln2_w, ln2_b,
           gate_W, gate_b, W1, b1, W2, b2):
    x2 = x.reshape(S, D)
    q16, k16, v = _qkv_proj(x2, Wq, bq, Wk, bk, Wv, bv)
    o = _attention(q16, k16, v)
    h = _proj_ln1(o, Wo, bo, ln1_w, ln1_b, x2)
    pos4, meta, vals = _route(h, gate_W, gate_b)
    pos = pos4.reshape(S * K)
    gathered = _dispatch(h, pos)
    eo = _gmm(meta, gathered, W1, b1, W2, b2)
    eo0, eo1 = _combine(eo, pos)
    out = _ln2(h, eo0, eo1, vals, ln2_w, ln2_b)
    return out.reshape(1, S, D)
